# Initial kernel scaffold; baseline (speedup 1.0000x reference)
import functools

import jax
import jax.numpy as jnp
from jax import lax
from jax.experimental import pallas as pl
from jax.experimental.pallas import tpu as pltpu
from jax.experimental.pallas import tpu_sc as plsc

N_NODES = 5000
D = 256
E = 160000

NC = 2
NS = 16
NT = NC * NS
CHUNK = 128
NCH = 40
E_PAD = NT * NCH * CHUNK
N_PAD = 5120
ROWS_PER_TILE = N_PAD // NS


def _sc_spmm(table, gidx3, sidx3, w3):
    mesh = plsc.VectorSubcoreMesh(core_axis_name="c", subcore_axis_name="s")

    @functools.partial(
        pl.kernel,
        out_type=jax.ShapeDtypeStruct((NC, N_PAD, D), jnp.float32),
        mesh=mesh,
        scratch_types=[
            pltpu.VMEM((NCH, CHUNK), jnp.int32),
            pltpu.VMEM((NCH, CHUNK), jnp.int32),
            pltpu.VMEM((NCH, CHUNK), jnp.float32),
            pltpu.VMEM((CHUNK, D), jnp.float32),
            pltpu.VMEM_SHARED((N_PAD, D), jnp.float32),
            pltpu.SemaphoreType.DMA,
        ],
    )
    def k(table_hbm, gidx_hbm, sidx_hbm, w_hbm, out_hbm,
          gidx_v, sidx_v, w_v, rows_v, acc, gsem):
        cid = lax.axis_index("c")
        sid = lax.axis_index("s")
        wid = cid * NS + sid

        pltpu.sync_copy(gidx_hbm.at[wid], gidx_v)
        pltpu.sync_copy(sidx_hbm.at[wid], sidx_v)
        pltpu.sync_copy(w_hbm.at[wid], w_v)

        @pl.loop(0, CHUNK)
        def _(r):
            for f in range(D // 16):
                rows_v.at[r, pl.ds(f * 16, 16)][...] = jnp.zeros((16,), jnp.float32)

        @pl.loop(0, ROWS_PER_TILE // 64)
        def _(z):
            base = sid * ROWS_PER_TILE + z * 64
            pltpu.sync_copy(rows_v.at[pl.ds(0, 64)], acc.at[pl.ds(base, 64)])

        plsc.subcore_barrier()

        @pl.loop(0, NCH)
        def _(j):
            pltpu.async_copy(table_hbm.at[gidx_v.at[j]], rows_v, gsem).wait()

            @pl.loop(0, CHUNK)
            def _(r):
                wspl = plsc.load_gather(
                    w_v,
                    [jnp.full((16,), j, jnp.int32), jnp.full((16,), r, jnp.int32)],
                )
                for f in range(D // 16):
                    sl = (r, pl.ds(f * 16, 16))
                    rows_v.at[*sl][...] = rows_v.at[*sl][...] * wspl

            pltpu.sync_copy(rows_v, acc.at[sidx_v.at[j]], add=True)

        plsc.subcore_barrier()

        base = sid * ROWS_PER_TILE
        pltpu.sync_copy(acc.at[pl.ds(base, ROWS_PER_TILE)],
                        out_hbm.at[cid, pl.ds(base, ROWS_PER_TILE)])

    return k(table, gidx3, sidx3, w3)


def _tc_dense(u_res_parts, i_res_parts, user_embedding, item_embedding, u_w, i_w):
    BLK = 1000
    grid = (N_NODES // BLK,)

    def body(ue_ref, pu_ref, ie_ref, pi_ref, uw_ref, iw_ref, ou_ref, oi_ref):
        xu = ue_ref[...] + pu_ref[0] + pu_ref[1]
        ou_ref[...] = jax.nn.sigmoid(
            jnp.dot(xu, uw_ref[...], preferred_element_type=jnp.float32))
        xi = ie_ref[...] + pi_ref[0] + pi_ref[1]
        oi_ref[...] = jax.nn.sigmoid(
            jnp.dot(xi, iw_ref[...], preferred_element_type=jnp.float32))

    emb_spec = pl.BlockSpec((BLK, D), lambda i: (i, 0))
    part_spec = pl.BlockSpec((NC, BLK, D), lambda i: (0, i, 0))
    w_spec = pl.BlockSpec((D, D), lambda i: (0, 0))

    return pl.pallas_call(
        body,
        grid=grid,
        in_specs=[emb_spec, part_spec, emb_spec, part_spec, w_spec, w_spec],
        out_specs=[emb_spec, emb_spec],
        out_shape=[
            jax.ShapeDtypeStruct((N_NODES, D), jnp.float32),
            jax.ShapeDtypeStruct((N_NODES, D), jnp.float32),
        ],
    )(user_embedding, u_res_parts, item_embedding, i_res_parts, u_w, i_w)


def kernel(user_embedding, item_embedding, edge_user, edge_item, edge_weight,
           u_w, i_w, ind_beh):
    eu = jnp.pad(edge_user.astype(jnp.int32), (0, E_PAD - E)).reshape(NT, NCH, CHUNK)
    ei = jnp.pad(edge_item.astype(jnp.int32), (0, E_PAD - E)).reshape(NT, NCH, CHUNK)
    w3 = jnp.pad(edge_weight, (0, E_PAD - E)).reshape(NT, NCH, CHUNK)

    part_u = _sc_spmm(item_embedding, ei, eu, w3)
    part_i = _sc_spmm(user_embedding, eu, ei, w3)

    u_emb, i_emb = _tc_dense(part_u, part_i, user_embedding, item_embedding,
                             u_w, i_w)
    return (u_emb, i_emb)


# speedup vs baseline: 2.8458x; 2.8458x over previous
import dataclasses
import functools

import jax
import jax.numpy as jnp
from jax import lax
from jax.experimental import pallas as pl
from jax.experimental.pallas import tpu as pltpu
from jax.experimental.pallas import tpu_sc as plsc

N_NODES = 5000
D = 256
E = 160000

NC = 2
NS = 16
NT = NC * NS
CHUNK = 128
NCH = 40
E_PAD = NT * NCH * CHUNK
STRIPE = 312
TAIL = N_NODES - NS * STRIPE


def _sc_spmm(table, resid, gidx3, sidx3, w3):
    mesh = plsc.VectorSubcoreMesh(core_axis_name="c", subcore_axis_name="s")

    cp = pltpu.CompilerParams()
    cp = dataclasses.replace(cp, needs_layout_passes=False,
                             use_tc_tiling_on_sc=False)

    @functools.partial(
        pl.kernel,
        compiler_params=cp,
        out_type=jax.ShapeDtypeStruct((NC, N_NODES, D), jnp.float32),
        mesh=mesh,
        scratch_types=[
            pltpu.VMEM((NCH, CHUNK), jnp.int32),
            pltpu.VMEM((NCH, CHUNK), jnp.int32),
            pltpu.VMEM((NCH, CHUNK), jnp.float32),
            pltpu.VMEM((CHUNK, D), jnp.float32),
            pltpu.VMEM_SHARED((N_NODES, D), jnp.float32),
            pltpu.SemaphoreType.DMA,
        ],
    )
    def k(table_hbm, resid_hbm, gidx_hbm, sidx_hbm, w_hbm, out_hbm,
          gidx_v, sidx_v, w_v, rows_v, acc, gsem):
        cid = lax.axis_index("c")
        sid = lax.axis_index("s")
        wid = cid * NS + sid

        pltpu.sync_copy(gidx_hbm.at[wid], gidx_v)
        pltpu.sync_copy(sidx_hbm.at[wid], sidx_v)
        pltpu.sync_copy(w_hbm.at[wid], w_v)

        base = sid * STRIPE

        @pl.when(cid == 0)
        def _():
            pltpu.sync_copy(resid_hbm.at[pl.ds(base, STRIPE)],
                            acc.at[pl.ds(base, STRIPE)])

            @pl.when(sid == 0)
            def _():
                pltpu.sync_copy(resid_hbm.at[pl.ds(NS * STRIPE, TAIL)],
                                acc.at[pl.ds(NS * STRIPE, TAIL)])

        @pl.when(cid == 1)
        def _():
            @pl.loop(0, 104)
            def _(r):
                for f in range(D // 16):
                    rows_v.at[r, pl.ds(f * 16, 16)][...] = jnp.zeros(
                        (16,), jnp.float32)

            @pl.loop(0, 3)
            def _(z):
                pltpu.sync_copy(rows_v.at[pl.ds(0, 104)],
                                acc.at[pl.ds(base + z * 104, 104)])

            @pl.when(sid == 0)
            def _():
                pltpu.sync_copy(rows_v.at[pl.ds(0, TAIL)],
                                acc.at[pl.ds(NS * STRIPE, TAIL)])

        plsc.subcore_barrier()

        @pl.loop(0, NCH)
        def _(j):
            pltpu.async_copy(table_hbm.at[gidx_v.at[j]], rows_v, gsem).wait()

            @pl.loop(0, CHUNK)
            def _(r):
                wspl = plsc.load_gather(
                    w_v,
                    [jnp.full((16,), j, jnp.int32),
                     jnp.full((16,), r, jnp.int32)],
                )
                for f in range(D // 16):
                    sl = (r, pl.ds(f * 16, 16))
                    rows_v.at[*sl][...] = rows_v.at[*sl][...] * wspl

            pltpu.sync_copy(rows_v, acc.at[sidx_v.at[j]], add=True)

        plsc.subcore_barrier()

        pltpu.sync_copy(acc.at[pl.ds(base, STRIPE)],
                        out_hbm.at[cid, pl.ds(base, STRIPE)])

        @pl.when(sid == 0)
        def _():
            pltpu.sync_copy(acc.at[pl.ds(NS * STRIPE, TAIL)],
                            out_hbm.at[cid, pl.ds(NS * STRIPE, TAIL)])

    return k(table, resid, gidx3, sidx3, w3)


def _tc_dense(u_parts, i_parts, u_w, i_w):
    BLK = 1000
    grid = (N_NODES // BLK,)

    def body(pu_ref, pi_ref, uw_ref, iw_ref, ou_ref, oi_ref):
        xu = pu_ref[0] + pu_ref[1]
        ou_ref[...] = jax.nn.sigmoid(
            jnp.dot(xu, uw_ref[...], preferred_element_type=jnp.float32))
        xi = pi_ref[0] + pi_ref[1]
        oi_ref[...] = jax.nn.sigmoid(
            jnp.dot(xi, iw_ref[...], preferred_element_type=jnp.float32))

    emb_spec = pl.BlockSpec((BLK, D), lambda i: (i, 0))
    part_spec = pl.BlockSpec((NC, BLK, D), lambda i: (0, i, 0))
    w_spec = pl.BlockSpec((D, D), lambda i: (0, 0))

    return pl.pallas_call(
        body,
        grid=grid,
        in_specs=[part_spec, part_spec, w_spec, w_spec],
        out_specs=[emb_spec, emb_spec],
        out_shape=[
            jax.ShapeDtypeStruct((N_NODES, D), jnp.float32),
            jax.ShapeDtypeStruct((N_NODES, D), jnp.float32),
        ],
    )(u_parts, i_parts, u_w, i_w)


def kernel(user_embedding, item_embedding, edge_user, edge_item, edge_weight,
           u_w, i_w, ind_beh):
    eu = jnp.pad(edge_user.astype(jnp.int32), (0, E_PAD - E)).reshape(NT, NCH, CHUNK)
    ei = jnp.pad(edge_item.astype(jnp.int32), (0, E_PAD - E)).reshape(NT, NCH, CHUNK)
    w3 = jnp.pad(edge_weight, (0, E_PAD - E)).reshape(NT, NCH, CHUNK)

    part_u = _sc_spmm(item_embedding, user_embedding, ei, eu, w3)
    part_i = _sc_spmm(user_embedding, item_embedding, eu, ei, w3)

    u_emb, i_emb = _tc_dense(part_u, part_i, u_w, i_w)
    return (u_emb, i_emb)


# speedup vs baseline: 3.9880x; 1.4014x over previous
import dataclasses
import functools

import jax
import jax.numpy as jnp
from jax import lax
from jax.experimental import pallas as pl
from jax.experimental.pallas import tpu as pltpu
from jax.experimental.pallas import tpu_sc as plsc

N_NODES = 5000
D = 256
E = 160000

NC = 2
NS = 16
CHUNK = 96
NCH = 108
E_PAD = NS * NCH * CHUNK
STRIPE = 312
TAIL = N_NODES - NS * STRIPE


def _sc_spmm(tables, resids, meta):
    mesh = plsc.VectorSubcoreMesh(core_axis_name="c", subcore_axis_name="s")

    cp = pltpu.CompilerParams()
    cp = dataclasses.replace(cp, needs_layout_passes=False,
                             use_tc_tiling_on_sc=False)

    @functools.partial(
        pl.kernel,
        compiler_params=cp,
        out_type=jax.ShapeDtypeStruct((NC, N_NODES, D), jnp.float32),
        mesh=mesh,
        scratch_types=[
            pltpu.VMEM((3, CHUNK), jnp.int32),
            pltpu.VMEM((3, CHUNK), jnp.int32),
            pltpu.VMEM((CHUNK, D), jnp.float32),
            pltpu.VMEM((CHUNK, D), jnp.float32),
            pltpu.VMEM_SHARED((N_NODES, D), jnp.float32),
            pltpu.SemaphoreType.DMA,
            pltpu.SemaphoreType.DMA,
        ],
    )
    def k(tables_hbm, resids_hbm, meta_hbm, out_hbm,
          ec_a, ec_b, rows_a, rows_b, acc, gsem, esem):
        cid = lax.axis_index("c")
        sid = lax.axis_index("s")

        base = sid * STRIPE
        rbase = cid * N_NODES
        pltpu.sync_copy(resids_hbm.at[pl.ds(rbase + base, STRIPE)],
                        acc.at[pl.ds(base, STRIPE)])

        @pl.when(sid == 0)
        def _():
            pltpu.sync_copy(resids_hbm.at[pl.ds(rbase + NS * STRIPE, TAIL)],
                            acc.at[pl.ds(NS * STRIPE, TAIL)])

        plsc.subcore_barrier()

        def scale(rows_v, ec_v):
            @pl.loop(0, CHUNK)
            def _(r):
                wspl = plsc.bitcast(
                    plsc.load_gather(
                        ec_v,
                        [jnp.full((16,), 2, jnp.int32),
                         jnp.full((16,), r, jnp.int32)],
                    ),
                    jnp.float32,
                )
                for f in range(D // 16):
                    sl = (r, pl.ds(f * 16, 16))
                    rows_v.at[*sl][...] = rows_v.at[*sl][...] * wspl

        def stage_meta(j, ec_v):
            pltpu.async_copy(meta_hbm.at[cid, sid, j], ec_v, esem)

        def wait_meta(ec_v):
            pltpu.make_async_copy(meta_hbm.at[0, 0, 0], ec_v, esem).wait()

        def gather(ec_v, rows_v):
            pltpu.async_copy(tables_hbm.at[ec_v.at[0]], rows_v, gsem)

        def wait_gather(rows_v):
            pltpu.make_async_copy(
                tables_hbm.at[ec_a.at[0]], rows_v, gsem).wait()

        def scatter_add(rows_v, ec_v):
            pltpu.sync_copy(rows_v, acc.at[ec_v.at[1]], add=True)

        stage_meta(0, ec_a)
        stage_meta(1, ec_b)
        wait_meta(ec_a)
        gather(ec_a, rows_a)
        wait_meta(ec_b)

        @pl.loop(0, NCH, step=2)
        def _(j):
            wait_gather(rows_a)
            gather(ec_b, rows_b)
            scale(rows_a, ec_a)
            scatter_add(rows_a, ec_a)

            @pl.when(j + 2 < NCH)
            def _():
                stage_meta(j + 2, ec_a)

            wait_gather(rows_b)

            @pl.when(j + 2 < NCH)
            def _():
                wait_meta(ec_a)
                gather(ec_a, rows_a)
            scale(rows_b, ec_b)
            scatter_add(rows_b, ec_b)

            @pl.when(j + 3 < NCH)
            def _():
                stage_meta(j + 3, ec_b)
                wait_meta(ec_b)

        plsc.subcore_barrier()

        pltpu.sync_copy(acc.at[pl.ds(base, STRIPE)],
                        out_hbm.at[cid, pl.ds(base, STRIPE)])

        @pl.when(sid == 0)
        def _():
            pltpu.sync_copy(acc.at[pl.ds(NS * STRIPE, TAIL)],
                            out_hbm.at[cid, pl.ds(NS * STRIPE, TAIL)])

    return k(tables, resids, meta)


def _tc_dense(parts, u_w, i_w):
    BLK = 1000
    grid = (N_NODES // BLK,)

    def body(pu_ref, pi_ref, uw_ref, iw_ref, ou_ref, oi_ref):
        ou_ref[...] = jax.nn.sigmoid(
            jnp.dot(pu_ref[0], uw_ref[...], preferred_element_type=jnp.float32))
        oi_ref[...] = jax.nn.sigmoid(
            jnp.dot(pi_ref[0], iw_ref[...], preferred_element_type=jnp.float32))

    emb_spec = pl.BlockSpec((BLK, D), lambda i: (i, 0))
    pu_spec = pl.BlockSpec((1, BLK, D), lambda i: (0, i, 0))
    pi_spec = pl.BlockSpec((1, BLK, D), lambda i: (1, i, 0))
    w_spec = pl.BlockSpec((D, D), lambda i: (0, 0))

    return pl.pallas_call(
        body,
        grid=grid,
        in_specs=[pu_spec, pi_spec, w_spec, w_spec],
        out_specs=[emb_spec, emb_spec],
        out_shape=[
            jax.ShapeDtypeStruct((N_NODES, D), jnp.float32),
            jax.ShapeDtypeStruct((N_NODES, D), jnp.float32),
        ],
    )(parts, parts, u_w, i_w)


def kernel(user_embedding, item_embedding, edge_user, edge_item, edge_weight,
           u_w, i_w, ind_beh):
    def prep(x):
        return jnp.pad(x, (0, E_PAD - E)).reshape(NS, NCH, 1, CHUNK)

    eu3 = prep(edge_user.astype(jnp.int32))
    ei3 = prep(edge_item.astype(jnp.int32))
    w3 = prep(jax.lax.bitcast_convert_type(edge_weight, jnp.int32))

    tables = jnp.concatenate([item_embedding, user_embedding], axis=0)
    resids = jnp.concatenate([user_embedding, item_embedding], axis=0)
    meta = jnp.stack([
        jnp.concatenate([ei3, eu3, w3], axis=2),
        jnp.concatenate([eu3 + N_NODES, ei3, w3], axis=2),
    ])

    parts = _sc_spmm(tables, resids, meta)
    u_emb, i_emb = _tc_dense(parts, u_w, i_w)
    return (u_emb, i_emb)


# speedup vs baseline: 3.9957x; 1.0019x over previous
import dataclasses
import functools

import jax
import jax.numpy as jnp
from jax import lax
from jax.experimental import pallas as pl
from jax.experimental.pallas import tpu as pltpu
from jax.experimental.pallas import tpu_sc as plsc

N_NODES = 5000
D = 256
E = 160000

NC = 2
NS = 16
NSUB = 4
CHUNK = 96
NCH = 108
E_PAD = NS * NCH * CHUNK
STRIPE = 312
TAIL = N_NODES - NS * STRIPE


def _sc_spmm(tables, resids, meta):
    mesh = plsc.VectorSubcoreMesh(core_axis_name="c", subcore_axis_name="s")

    cp = pltpu.CompilerParams()
    cp = dataclasses.replace(cp, needs_layout_passes=False,
                             use_tc_tiling_on_sc=False)

    @functools.partial(
        pl.kernel,
        compiler_params=cp,
        out_type=jax.ShapeDtypeStruct((NC, N_NODES, D), jnp.float32),
        mesh=mesh,
        scratch_types=[
            pltpu.VMEM((3, CHUNK), jnp.int32),
            pltpu.VMEM((3, CHUNK), jnp.int32),
            pltpu.VMEM((CHUNK, D), jnp.float32),
            pltpu.VMEM((CHUNK, D), jnp.float32),
            pltpu.VMEM_SHARED((N_NODES, D), jnp.float32),
            pltpu.SemaphoreType.DMA,
            pltpu.SemaphoreType.DMA,
        ],
    )
    def k(tables_hbm, resids_hbm, meta_hbm, out_hbm,
          ec_a, ec_b, rows_a, rows_b, acc, gsem, esem):
        cid = lax.axis_index("c")
        sid = lax.axis_index("s")

        base = sid * STRIPE
        rbase = cid * N_NODES
        pltpu.sync_copy(resids_hbm.at[pl.ds(rbase + base, STRIPE)],
                        acc.at[pl.ds(base, STRIPE)])

        @pl.when(sid == 0)
        def _():
            pltpu.sync_copy(resids_hbm.at[pl.ds(rbase + NS * STRIPE, TAIL)],
                            acc.at[pl.ds(NS * STRIPE, TAIL)])

        plsc.subcore_barrier()

        def scale(rows_v, ec_v):
            @pl.loop(0, CHUNK)
            def _(r):
                wspl = plsc.bitcast(
                    plsc.load_gather(
                        ec_v,
                        [jnp.full((16,), 2, jnp.int32),
                         jnp.full((16,), r, jnp.int32)],
                    ),
                    jnp.float32,
                )
                for f in range(D // 16):
                    sl = (r, pl.ds(f * 16, 16))
                    rows_v.at[*sl][...] = rows_v.at[*sl][...] * wspl

        def stage_meta(j, ec_v):
            pltpu.async_copy(meta_hbm.at[cid, sid, j], ec_v, esem)

        def wait_meta(ec_v):
            pltpu.make_async_copy(meta_hbm.at[0, 0, 0], ec_v, esem).wait()

        SUB = CHUNK // NSUB

        def gather(ec_v, rows_v):
            for sft in range(NSUB):
                pltpu.async_copy(
                    tables_hbm.at[ec_v.at[0, pl.ds(SUB * sft, SUB)]],
                    rows_v.at[pl.ds(SUB * sft, SUB)], gsem)

        def wait_gather(rows_v):
            for sft in range(NSUB):
                pltpu.make_async_copy(
                    tables_hbm.at[ec_a.at[0, pl.ds(0, SUB)]],
                    rows_v.at[pl.ds(SUB * sft, SUB)], gsem).wait()

        def scatter_add(rows_v, ec_v):
            pltpu.sync_copy(rows_v, acc.at[ec_v.at[1]], add=True)

        stage_meta(0, ec_a)
        stage_meta(1, ec_b)
        wait_meta(ec_a)
        gather(ec_a, rows_a)
        wait_meta(ec_b)

        @pl.loop(0, NCH, step=2)
        def _(j):
            wait_gather(rows_a)
            gather(ec_b, rows_b)
            scale(rows_a, ec_a)
            scatter_add(rows_a, ec_a)

            @pl.when(j + 2 < NCH)
            def _():
                stage_meta(j + 2, ec_a)

            wait_gather(rows_b)

            @pl.when(j + 2 < NCH)
            def _():
                wait_meta(ec_a)
                gather(ec_a, rows_a)
            scale(rows_b, ec_b)
            scatter_add(rows_b, ec_b)

            @pl.when(j + 3 < NCH)
            def _():
                stage_meta(j + 3, ec_b)
                wait_meta(ec_b)

        plsc.subcore_barrier()

        pltpu.sync_copy(acc.at[pl.ds(base, STRIPE)],
                        out_hbm.at[cid, pl.ds(base, STRIPE)])

        @pl.when(sid == 0)
        def _():
            pltpu.sync_copy(acc.at[pl.ds(NS * STRIPE, TAIL)],
                            out_hbm.at[cid, pl.ds(NS * STRIPE, TAIL)])

    return k(tables, resids, meta)


def _tc_dense(parts, u_w, i_w):
    BLK = 1000
    grid = (N_NODES // BLK,)

    def body(pu_ref, pi_ref, uw_ref, iw_ref, ou_ref, oi_ref):
        ou_ref[...] = jax.nn.sigmoid(
            jnp.dot(pu_ref[0], uw_ref[...], preferred_element_type=jnp.float32))
        oi_ref[...] = jax.nn.sigmoid(
            jnp.dot(pi_ref[0], iw_ref[...], preferred_element_type=jnp.float32))

    emb_spec = pl.BlockSpec((BLK, D), lambda i: (i, 0))
    pu_spec = pl.BlockSpec((1, BLK, D), lambda i: (0, i, 0))
    pi_spec = pl.BlockSpec((1, BLK, D), lambda i: (1, i, 0))
    w_spec = pl.BlockSpec((D, D), lambda i: (0, 0))

    return pl.pallas_call(
        body,
        grid=grid,
        in_specs=[pu_spec, pi_spec, w_spec, w_spec],
        out_specs=[emb_spec, emb_spec],
        out_shape=[
            jax.ShapeDtypeStruct((N_NODES, D), jnp.float32),
            jax.ShapeDtypeStruct((N_NODES, D), jnp.float32),
        ],
    )(parts, parts, u_w, i_w)


def kernel(user_embedding, item_embedding, edge_user, edge_item, edge_weight,
           u_w, i_w, ind_beh):
    def prep(x):
        return jnp.pad(x, (0, E_PAD - E)).reshape(NS, NCH, 1, CHUNK)

    eu3 = prep(edge_user.astype(jnp.int32))
    ei3 = prep(edge_item.astype(jnp.int32))
    w3 = prep(jax.lax.bitcast_convert_type(edge_weight, jnp.int32))

    tables = jnp.concatenate([item_embedding, user_embedding], axis=0)
    resids = jnp.concatenate([user_embedding, item_embedding], axis=0)
    meta = jnp.stack([
        jnp.concatenate([ei3, eu3, w3], axis=2),
        jnp.concatenate([eu3 + N_NODES, ei3, w3], axis=2),
    ])

    parts = _sc_spmm(tables, resids, meta)
    u_emb, i_emb = _tc_dense(parts, u_w, i_w)
    return (u_emb, i_emb)


# speedup vs baseline: 6.4726x; 1.6199x over previous
import dataclasses
import functools

import jax
import jax.numpy as jnp
from jax import lax
from jax.experimental import pallas as pl
from jax.experimental.pallas import tpu as pltpu
from jax.experimental.pallas import tpu_sc as plsc

N_NODES = 5000
D = 256
E = 160000

NC = 2
NS = 16
EPT = E // NS
CHUNK = 80
NCH = EPT // CHUNK
STRIPE = 312
TAIL = N_NODES - NS * STRIPE


def _sc_spmm(user_embedding, item_embedding, edge_user, edge_item,
             edge_weight):
    mesh = plsc.VectorSubcoreMesh(core_axis_name="c", subcore_axis_name="s")

    cp = pltpu.CompilerParams()
    cp = dataclasses.replace(cp, needs_layout_passes=False,
                             use_tc_tiling_on_sc=False)

    @functools.partial(
        pl.kernel,
        compiler_params=cp,
        out_type=jax.ShapeDtypeStruct((NC, N_NODES, D), jnp.float32),
        mesh=mesh,
        scratch_types=[
            pltpu.VMEM((CHUNK,), jnp.int32),
            pltpu.VMEM((CHUNK,), jnp.int32),
            pltpu.VMEM((CHUNK,), jnp.int32),
            pltpu.VMEM((CHUNK,), jnp.int32),
            pltpu.VMEM((CHUNK,), jnp.float32),
            pltpu.VMEM((CHUNK,), jnp.float32),
            pltpu.VMEM((CHUNK, D), jnp.float32),
            pltpu.VMEM((CHUNK, D), jnp.float32),
            pltpu.VMEM_SHARED((N_NODES, D), jnp.float32),
            pltpu.SemaphoreType.DMA,
            pltpu.SemaphoreType.DMA,
        ],
    )
    def k(ue_hbm, ie_hbm, eu_hbm, ei_hbm, w_hbm, out_hbm,
          gi_a, gi_b, si_a, si_b, w_a, w_b, rows_a, rows_b, acc, gsem, esem):
        cid = lax.axis_index("c")
        sid = lax.axis_index("s")

        base = sid * STRIPE

        @pl.when(cid == 0)
        def _():
            pltpu.sync_copy(ue_hbm.at[pl.ds(base, STRIPE)],
                            acc.at[pl.ds(base, STRIPE)])

            @pl.when(sid == 0)
            def _():
                pltpu.sync_copy(ue_hbm.at[pl.ds(NS * STRIPE, TAIL)],
                                acc.at[pl.ds(NS * STRIPE, TAIL)])

        @pl.when(cid == 1)
        def _():
            pltpu.sync_copy(ie_hbm.at[pl.ds(base, STRIPE)],
                            acc.at[pl.ds(base, STRIPE)])

            @pl.when(sid == 0)
            def _():
                pltpu.sync_copy(ie_hbm.at[pl.ds(NS * STRIPE, TAIL)],
                                acc.at[pl.ds(NS * STRIPE, TAIL)])

        plsc.subcore_barrier()

        ebase = sid * EPT

        def scale(rows_v, w_v):
            @pl.loop(0, CHUNK)
            def _(r):
                wspl = plsc.load_gather(w_v, [jnp.full((16,), r, jnp.int32)])
                for f in range(D // 16):
                    sl = (r, pl.ds(f * 16, 16))
                    rows_v.at[*sl][...] = rows_v.at[*sl][...] * wspl

        def stage_meta(j, gi_v, si_v, w_v):
            sl = pl.ds(ebase + j * CHUNK, CHUNK)
            @pl.when(cid == 0)
            def _():
                pltpu.async_copy(ei_hbm.at[sl], gi_v, esem)
                pltpu.async_copy(eu_hbm.at[sl], si_v, esem)

            @pl.when(cid == 1)
            def _():
                pltpu.async_copy(eu_hbm.at[sl], gi_v, esem)
                pltpu.async_copy(ei_hbm.at[sl], si_v, esem)
            pltpu.async_copy(w_hbm.at[sl], w_v, esem)

        def wait_meta(gi_v, si_v, w_v):
            sl = pl.ds(0, CHUNK)
            pltpu.make_async_copy(eu_hbm.at[sl], gi_v, esem).wait()
            pltpu.make_async_copy(eu_hbm.at[sl], si_v, esem).wait()
            pltpu.make_async_copy(w_hbm.at[sl], w_v, esem).wait()

        def gather(gi_v, rows_v):
            @pl.when(cid == 0)
            def _():
                pltpu.async_copy(ie_hbm.at[gi_v], rows_v, gsem)

            @pl.when(cid == 1)
            def _():
                pltpu.async_copy(ue_hbm.at[gi_v], rows_v, gsem)

        def wait_gather(rows_v):
            pltpu.make_async_copy(ie_hbm.at[gi_a], rows_v, gsem).wait()

        def scatter_add(rows_v, si_v):
            pltpu.sync_copy(rows_v, acc.at[si_v], add=True)

        stage_meta(0, gi_a, si_a, w_a)
        stage_meta(1, gi_b, si_b, w_b)
        wait_meta(gi_a, si_a, w_a)
        gather(gi_a, rows_a)
        wait_meta(gi_b, si_b, w_b)

        @pl.loop(0, NCH - 1, step=2)
        def _(j):
            wait_gather(rows_a)
            gather(gi_b, rows_b)
            scale(rows_a, w_a)
            scatter_add(rows_a, si_a)

            @pl.when(j + 2 < NCH)
            def _():
                stage_meta(j + 2, gi_a, si_a, w_a)

            wait_gather(rows_b)

            @pl.when(j + 2 < NCH)
            def _():
                wait_meta(gi_a, si_a, w_a)
                gather(gi_a, rows_a)
            scale(rows_b, w_b)
            scatter_add(rows_b, si_b)

            @pl.when(j + 3 < NCH)
            def _():
                stage_meta(j + 3, gi_b, si_b, w_b)
                wait_meta(gi_b, si_b, w_b)

        wait_gather(rows_a)
        scale(rows_a, w_a)
        scatter_add(rows_a, si_a)

        plsc.subcore_barrier()

        pltpu.sync_copy(acc.at[pl.ds(base, STRIPE)],
                        out_hbm.at[cid, pl.ds(base, STRIPE)])

        @pl.when(sid == 0)
        def _():
            pltpu.sync_copy(acc.at[pl.ds(NS * STRIPE, TAIL)],
                            out_hbm.at[cid, pl.ds(NS * STRIPE, TAIL)])

    return k(user_embedding, item_embedding, edge_user, edge_item,
             edge_weight)


def _tc_dense(parts, u_w, i_w):
    BLK = 1000
    grid = (N_NODES // BLK,)

    def body(pu_ref, pi_ref, uw_ref, iw_ref, ou_ref, oi_ref):
        ou_ref[...] = jax.nn.sigmoid(
            jnp.dot(pu_ref[0], uw_ref[...], preferred_element_type=jnp.float32))
        oi_ref[...] = jax.nn.sigmoid(
            jnp.dot(pi_ref[0], iw_ref[...], preferred_element_type=jnp.float32))

    emb_spec = pl.BlockSpec((BLK, D), lambda i: (i, 0))
    pu_spec = pl.BlockSpec((1, BLK, D), lambda i: (0, i, 0))
    pi_spec = pl.BlockSpec((1, BLK, D), lambda i: (1, i, 0))
    w_spec = pl.BlockSpec((D, D), lambda i: (0, 0))

    return pl.pallas_call(
        body,
        grid=grid,
        in_specs=[pu_spec, pi_spec, w_spec, w_spec],
        out_specs=[emb_spec, emb_spec],
        out_shape=[
            jax.ShapeDtypeStruct((N_NODES, D), jnp.float32),
            jax.ShapeDtypeStruct((N_NODES, D), jnp.float32),
        ],
    )(parts, parts, u_w, i_w)


def kernel(user_embedding, item_embedding, edge_user, edge_item, edge_weight,
           u_w, i_w, ind_beh):
    parts = _sc_spmm(user_embedding, item_embedding,
                     edge_user.astype(jnp.int32), edge_item.astype(jnp.int32),
                     edge_weight)
    u_emb, i_emb = _tc_dense(parts, u_w, i_w)
    return (u_emb, i_emb)
